# baseline (speedup 1.0000x reference)
import functools

import jax
import jax.numpy as jnp
from jax.experimental import pallas as pl
from jax.experimental.pallas import tpu as pltpu

_EPS = 1e-5


def _h(i, bx, rf, h1, h2, h3, w1_ref, b1_ref, w2_ref, b2_ref):
    b1i = b1_ref[i : i + 1, :]
    if i == 0:
        w1a = w1_ref[0, :128, :].astype(jnp.bfloat16)
        w1b = w1_ref[0, 128:, :].astype(jnp.bfloat16)
        pre = (
            jnp.dot(bx[...].astype(jnp.bfloat16), w1a,
                    preferred_element_type=jnp.float32)
            + jnp.dot(rf[...].astype(jnp.bfloat16), w1b,
                      preferred_element_type=jnp.float32)
            + b1i
        )
    else:
        f = (h1, h2, h3)[i - 1][...].astype(jnp.bfloat16)
        pre = jnp.dot(f, w1_ref[i].astype(jnp.bfloat16),
                      preferred_element_type=jnp.float32) + b1i
    g = jnp.maximum(pre, 0.0).astype(jnp.bfloat16)
    return jnp.dot(g, w2_ref[i].astype(jnp.bfloat16),
                   preferred_element_type=jnp.float32) + b2_ref[i : i + 1, :]


def _pass1_kernel(n_rows, n_blocks,
                  bx, rf, h1, h2, h3, w1, b1, w2, b2, gamma, beta,
                  hstore, avec_out, cvec_out, s1, s2):
    step = pl.program_id(0)

    @pl.when(step == 0)
    def _init():
        s1[...] = jnp.zeros_like(s1)
        s2[...] = jnp.zeros_like(s2)

    blk = hstore.shape[1]
    ones = jnp.ones((8, blk), dtype=jnp.bfloat16)
    for i in range(4):
        h = _h(i, bx, rf, h1, h2, h3, w1, b1, w2, b2)
        hb = h.astype(jnp.bfloat16)
        hstore[i] = hb
        s1[i] = s1[i] + jnp.dot(ones, hb, preferred_element_type=jnp.float32)
        s2[i] = s2[i] + jnp.dot(ones, hb * hb,
                                preferred_element_type=jnp.float32)

    @pl.when(step == n_blocks - 1)
    def _finalize():
        n = jnp.float32(n_rows)
        mu = s1[:, 0, :] / n
        var = s2[:, 0, :] / n - mu * mu
        a = gamma[...] * jax.lax.rsqrt(var + _EPS)
        avec_out[...] = a
        cvec_out[...] = jnp.sum(beta[...] - mu * a, axis=0, keepdims=True)


def _pass2_kernel(hstore, avec, cvec, out_ref):
    a = avec[...]
    acc = jnp.broadcast_to(cvec[...], out_ref.shape)
    for i in range(4):
        acc = acc + hstore[i].astype(jnp.float32) * a[i : i + 1, :]
    out_ref[...] = acc


def _pick_block(n, candidates):
    for b in candidates:
        if n % b == 0:
            return b
    return n


@jax.jit
def kernel(x, edges, batch_x, rand_feature, hop1_feature, hop2_feature,
           hop3_feature, W1, b1, W2, b2, gamma, beta):
    del x, edges
    n, d_in = batch_x.shape
    rw = rand_feature.shape[1]
    h2_dim = W2.shape[-1]
    blk = _pick_block(n, (4000, 2000, 1000, 200, 40, 8))
    n_blocks = n // blk
    blk2 = _pick_block(n, (10000, 4000, 2000, 1000, 200, 40, 8))
    n_blocks2 = n // blk2

    row_specs = [
        pl.BlockSpec((blk, d_in), lambda i: (i, 0)),
        pl.BlockSpec((blk, rw), lambda i: (i, 0)),
        pl.BlockSpec((blk, d_in + rw), lambda i: (i, 0)),
        pl.BlockSpec((blk, d_in + rw), lambda i: (i, 0)),
        pl.BlockSpec((blk, d_in + rw), lambda i: (i, 0)),
    ]
    w1_spec = pl.BlockSpec(W1.shape, lambda i: (0, 0, 0))
    vec_spec = pl.BlockSpec((4, h2_dim), lambda i: (0, 0))
    w2_spec = pl.BlockSpec(W2.shape, lambda i: (0, 0, 0))
    hstore_spec = pl.BlockSpec((4, blk, h2_dim), lambda i: (0, i, 0))

    hstore, avec, cvec = pl.pallas_call(
        functools.partial(_pass1_kernel, n, n_blocks),
        grid=(n_blocks,),
        in_specs=row_specs + [w1_spec, vec_spec, w2_spec, vec_spec,
                              vec_spec, vec_spec],
        out_specs=[
            hstore_spec,
            pl.BlockSpec((4, h2_dim), lambda i: (0, 0)),
            pl.BlockSpec((1, h2_dim), lambda i: (0, 0)),
        ],
        out_shape=[
            jax.ShapeDtypeStruct((4, n, h2_dim), jnp.bfloat16),
            jax.ShapeDtypeStruct((4, h2_dim), jnp.float32),
            jax.ShapeDtypeStruct((1, h2_dim), jnp.float32),
        ],
        scratch_shapes=[
            pltpu.VMEM((4, 8, h2_dim), jnp.float32),
            pltpu.VMEM((4, 8, h2_dim), jnp.float32),
        ],
        compiler_params=pltpu.CompilerParams(
            dimension_semantics=("arbitrary",)),
    )(batch_x, rand_feature, hop1_feature, hop2_feature, hop3_feature,
      W1, b1, W2, b2, gamma, beta)

    out = pl.pallas_call(
        _pass2_kernel,
        grid=(n_blocks2,),
        in_specs=[
            pl.BlockSpec((4, blk2, h2_dim), lambda i: (0, i, 0)),
            pl.BlockSpec((4, h2_dim), lambda i: (0, 0)),
            pl.BlockSpec((1, h2_dim), lambda i: (0, 0)),
        ],
        out_specs=pl.BlockSpec((blk2, h2_dim), lambda i: (i, 0)),
        out_shape=jax.ShapeDtypeStruct((n, h2_dim), jnp.float32),
        compiler_params=pltpu.CompilerParams(
            dimension_semantics=("arbitrary",)),
    )(hstore, avec, cvec)

    return out


# speedup vs baseline: 1.0907x; 1.0907x over previous
import functools

import jax
import jax.numpy as jnp
from jax.experimental import pallas as pl
from jax.experimental.pallas import tpu as pltpu

_EPS = 1e-5


def _h(i, bx, rft, h1, h2, h3, w1_ref, w1bp_ref, b1_ref, w2_ref, b2_ref):
    b1i = b1_ref[i : i + 1, :]
    if i == 0:
        w1a = w1_ref[0, :128, :].astype(jnp.bfloat16)
        pre = (
            jnp.dot(bx[...].astype(jnp.bfloat16), w1a,
                    preferred_element_type=jnp.float32)
            + jax.lax.dot_general(
                rft[...].astype(jnp.bfloat16),
                w1bp_ref[...].astype(jnp.bfloat16),
                (((0,), (0,)), ((), ())),
                preferred_element_type=jnp.float32)
            + b1i
        )
    else:
        f = (h1, h2, h3)[i - 1][...].astype(jnp.bfloat16)
        pre = jnp.dot(f, w1_ref[i].astype(jnp.bfloat16),
                      preferred_element_type=jnp.float32) + b1i
    g = jnp.maximum(pre, 0.0).astype(jnp.bfloat16)
    return jnp.dot(g, w2_ref[i].astype(jnp.bfloat16),
                   preferred_element_type=jnp.float32) + b2_ref[i : i + 1, :]


def _pass1_kernel(n_rows, n_blocks,
                  bx, rft, h1, h2, h3, w1, w1bp, b1, w2, b2, gamma, beta,
                  hstore, avec_out, cvec_out, s1, s2):
    step = pl.program_id(0)

    @pl.when(step == 0)
    def _init():
        s1[...] = jnp.zeros_like(s1)
        s2[...] = jnp.zeros_like(s2)

    blk = hstore.shape[1]
    ones = jnp.ones((8, blk), dtype=jnp.bfloat16)
    hbs = []
    for i in range(4):
        hb = _h(i, bx, rft, h1, h2, h3, w1, w1bp, b1, w2,
                b2).astype(jnp.bfloat16)
        hstore[i] = hb
        hbs.append(hb)

    @pl.when(step < n_blocks - 1)
    def _stats_full():
        for i in range(4):
            s1[i] = s1[i] + jnp.dot(ones, hbs[i],
                                    preferred_element_type=jnp.float32)
            s2[i] = s2[i] + jnp.dot(ones, hbs[i] * hbs[i],
                                    preferred_element_type=jnp.float32)

    @pl.when(step == n_blocks - 1)
    def _stats_masked():
        valid = n_rows - step * blk
        row = jax.lax.broadcasted_iota(jnp.int32, (blk, hstore.shape[2]), 0)
        keep = row < valid
        for i in range(4):
            hm = jnp.where(keep, hbs[i], jnp.bfloat16(0.0))
            s1[i] = s1[i] + jnp.dot(ones, hm,
                                    preferred_element_type=jnp.float32)
            s2[i] = s2[i] + jnp.dot(ones, hm * hm,
                                    preferred_element_type=jnp.float32)

        n = jnp.float32(n_rows)
        mu = s1[:, 0, :] / n
        var = s2[:, 0, :] / n - mu * mu
        a = gamma[...] * jax.lax.rsqrt(var + _EPS)
        avec_out[...] = a
        cvec_out[...] = jnp.sum(beta[...] - mu * a, axis=0, keepdims=True)


def _pass2_kernel(hstore, avec, cvec, out_ref):
    a = avec[...]
    acc = jnp.broadcast_to(cvec[...], out_ref.shape)
    for i in range(4):
        acc = acc + hstore[i].astype(jnp.float32) * a[i : i + 1, :]
    out_ref[...] = acc


def _pick_block(n, candidates):
    for b in candidates:
        if n % b == 0:
            return b
    return n


@jax.jit
def kernel(x, edges, batch_x, rand_feature, hop1_feature, hop2_feature,
           hop3_feature, W1, b1, W2, b2, gamma, beta):
    del x, edges
    n, d_in = batch_x.shape
    rw = rand_feature.shape[1]
    h2_dim = W2.shape[-1]
    blk = min(4096, ((n + 127) // 128) * 128)
    n_blocks = -(-n // blk)
    blk2 = _pick_block(n, (10000, 4000, 2000, 1000, 200, 40, 8))
    n_blocks2 = n // blk2

    rft = jnp.pad(rand_feature.T, ((0, 8 - rw), (0, n_blocks * blk - n)))
    w1bp = jnp.pad(W1[0, d_in:, :], ((0, 8 - rw), (0, 0)))

    row_specs = [
        pl.BlockSpec((blk, d_in), lambda i: (i, 0)),
        pl.BlockSpec((8, blk), lambda i: (0, i)),
        pl.BlockSpec((blk, d_in + rw), lambda i: (i, 0)),
        pl.BlockSpec((blk, d_in + rw), lambda i: (i, 0)),
        pl.BlockSpec((blk, d_in + rw), lambda i: (i, 0)),
    ]
    w1_spec = pl.BlockSpec(W1.shape, lambda i: (0, 0, 0))
    vec_spec = pl.BlockSpec((4, h2_dim), lambda i: (0, 0))
    w2_spec = pl.BlockSpec(W2.shape, lambda i: (0, 0, 0))
    hstore_spec = pl.BlockSpec((4, blk, h2_dim), lambda i: (0, i, 0))

    hstore, avec, cvec = pl.pallas_call(
        functools.partial(_pass1_kernel, n, n_blocks),
        grid=(n_blocks,),
        in_specs=row_specs + [w1_spec, pl.BlockSpec((8, h2_dim),
                                                    lambda i: (0, 0)),
                              vec_spec, w2_spec, vec_spec,
                              vec_spec, vec_spec],
        out_specs=[
            hstore_spec,
            pl.BlockSpec((4, h2_dim), lambda i: (0, 0)),
            pl.BlockSpec((1, h2_dim), lambda i: (0, 0)),
        ],
        out_shape=[
            jax.ShapeDtypeStruct((4, n, h2_dim), jnp.bfloat16),
            jax.ShapeDtypeStruct((4, h2_dim), jnp.float32),
            jax.ShapeDtypeStruct((1, h2_dim), jnp.float32),
        ],
        scratch_shapes=[
            pltpu.VMEM((4, 8, h2_dim), jnp.float32),
            pltpu.VMEM((4, 8, h2_dim), jnp.float32),
        ],
        compiler_params=pltpu.CompilerParams(
            dimension_semantics=("arbitrary",)),
    )(batch_x, rft, hop1_feature, hop2_feature, hop3_feature,
      W1, w1bp, b1, W2, b2, gamma, beta)

    out = pl.pallas_call(
        _pass2_kernel,
        grid=(n_blocks2,),
        in_specs=[
            pl.BlockSpec((4, blk2, h2_dim), lambda i: (0, i, 0)),
            pl.BlockSpec((4, h2_dim), lambda i: (0, 0)),
            pl.BlockSpec((1, h2_dim), lambda i: (0, 0)),
        ],
        out_specs=pl.BlockSpec((blk2, h2_dim), lambda i: (i, 0)),
        out_shape=jax.ShapeDtypeStruct((n, h2_dim), jnp.float32),
        compiler_params=pltpu.CompilerParams(
            dimension_semantics=("arbitrary",)),
    )(hstore, avec, cvec)

    return out
